# baseline (speedup 1.0000x reference)
import functools

import jax
import jax.numpy as jnp
from jax import lax
from jax.experimental import pallas as pl
from jax.experimental.pallas import tpu as pltpu
from jax.experimental.pallas import tpu_sc as plsc

N_EMB = 1000
ROWS = 4096 * 20
NC, NS, L = 2, 16, 16
ROWS_PER_T = ROWS // (NC * NS)
Z = 640_000
SC_FLOATS = ROWS // NC * N_EMB
Z_PER_T = SC_FLOATS // (NS * Z)
K = 128
NK = ROWS_PER_T // K


def _one_hot_sc(idx_hbm, zeros_hbm):
    mesh = plsc.VectorSubcoreMesh(
        core_axis_name="c", subcore_axis_name="s", num_cores=NC, num_subcores=NS
    )

    @functools.partial(
        pl.kernel,
        out_type=jax.ShapeDtypeStruct((ROWS * N_EMB,), jnp.float32),
        mesh=mesh,
        scratch_types=[
            pltpu.VMEM_SHARED((Z,), jnp.float32),
            pltpu.VMEM((ROWS_PER_T,), jnp.int32),
            pltpu.VMEM((NK, K), jnp.int32),
            pltpu.VMEM((K,), jnp.float32),
            pltpu.SemaphoreType.DMA,
            pltpu.SemaphoreType.DMA,
        ],
    )
    def body(idx_ref, zeros_ref, out_ref, zblk, idx_v, pos_v, ones_b, sem_z, sem_s):
        cid = lax.axis_index("c")
        sid = lax.axis_index("s")
        tile_row0 = (cid * NS + sid) * ROWS_PER_T

        @pl.when(sid == 0)
        def _():
            pltpu.sync_copy(zeros_ref, zblk)

        plsc.subcore_barrier()

        sc_base = cid * SC_FLOATS
        for k in range(Z_PER_T):
            dst = out_ref.at[pl.ds(sc_base + (sid * Z_PER_T + k) * Z, Z)]
            pltpu.async_copy(zblk, dst, sem_z)

        pltpu.sync_copy(idx_ref.at[pl.ds(tile_row0, ROWS_PER_T)], idx_v)
        lane = lax.iota(jnp.int32, L)
        ones_v = jnp.full((L,), 1.0, jnp.float32)
        for q in range(K // L):
            ones_b[pl.ds(q * L, L)] = ones_v

        def fill_pos(j, carry):
            for q in range(K // L):
                g = j * K + q * L
                idx16 = idx_v[pl.ds(g, L)]
                pos_v[j, pl.ds(q * L, L)] = (tile_row0 + g + lane) * N_EMB + idx16
            return carry

        lax.fori_loop(0, NK, fill_pos, 0)

        for k in range(Z_PER_T):
            pltpu.make_async_copy(zblk, out_ref.at[pl.ds(0, Z)], sem_z).wait()
        plsc.subcore_barrier()

        for j in range(NK):
            pltpu.async_copy(ones_b, out_ref.at[pos_v.at[j]], sem_s)
        for j in range(NK):
            pltpu.make_async_copy(ones_b, out_ref.at[pos_v.at[0]], sem_s).wait()

    return body(idx_hbm, zeros_hbm)


@jax.jit
def kernel(inputs):
    idx = inputs.reshape(-1).astype(jnp.int32)
    zeros = jnp.zeros((Z,), jnp.float32)
    flat = _one_hot_sc(idx, zeros)
    return flat.reshape(4096, 20, N_EMB)


# speedup vs baseline: 8.1200x; 8.1200x over previous
import functools

import jax
import jax.numpy as jnp
from jax.experimental import pallas as pl
from jax.experimental.pallas import tpu as pltpu

N_EMB = 1000
B = 4096
S = 20
KBLK = 200


def _body(idx_ref, out_ref):
    kb = pl.program_id(1)
    kk = jax.lax.broadcasted_iota(jnp.int32, (1, KBLK, B), 1) + kb * KBLK
    idx = idx_ref[...]
    out_ref[...] = (kk == idx).astype(jnp.float32)


@jax.jit
def kernel(inputs):
    idx_t = inputs.astype(jnp.int32).T.reshape(S, 1, B)
    out_t = pl.pallas_call(
        _body,
        out_shape=jax.ShapeDtypeStruct((S, N_EMB, B), jnp.float32),
        grid=(S, N_EMB // KBLK),
        in_specs=[pl.BlockSpec((1, 1, B), lambda j, kb: (j, 0, 0))],
        out_specs=pl.BlockSpec((1, KBLK, B), lambda j, kb: (j, kb, 0)),
    )(idx_t)
    return jnp.transpose(out_t, (2, 0, 1))


# speedup vs baseline: 8.2091x; 1.0110x over previous
import functools

import jax
import jax.numpy as jnp
from jax.experimental import pallas as pl
from jax.experimental.pallas import tpu as pltpu

N_EMB = 1000
B = 4096
S = 20
KBLK = 1000


def _body(idx_ref, out_ref):
    kb = pl.program_id(1)
    kk = jax.lax.broadcasted_iota(jnp.int32, (1, KBLK, B), 1) + kb * KBLK
    idx = idx_ref[...]
    out_ref[...] = (kk == idx).astype(jnp.float32)


@jax.jit
def kernel(inputs):
    idx_t = inputs.astype(jnp.int32).T.reshape(S, 1, B)
    out_t = pl.pallas_call(
        _body,
        out_shape=jax.ShapeDtypeStruct((S, N_EMB, B), jnp.float32),
        grid=(S, N_EMB // KBLK),
        in_specs=[pl.BlockSpec((1, 1, B), lambda j, kb: (j, 0, 0))],
        out_specs=pl.BlockSpec((1, KBLK, B), lambda j, kb: (j, kb, 0)),
    )(idx_t)
    return jnp.transpose(out_t, (2, 0, 1))
